# baseline (speedup 1.0000x reference)
import jax
import jax.numpy as jnp
from jax import lax
from jax.experimental import pallas as pl
from jax.experimental.pallas import tpu as pltpu
from jax.experimental.pallas import tpu_sc as plsc

B, L, V, D, TREE = 16, 2048, 50000, 512, 4
NC, NS, LANES = 2, 16, 16
NW = NC * NS
LW = L // NW
ROWS_W = LW * B
G_TS = 2
G_ROWS = G_TS * B
N_G = LW // G_TS
NBUF = 4
CHUNKS = D // LANES


def _body(tokens_hbm, trees_hbm, lengths_hbm, table_hbm, w_hbm, out_hbm,
          len_v, key_buf, tok_v, trees_v, wv, idx_buf, m_buf, coef_buf,
          buf_0, buf_1, buf_2, buf_3, sem_0, sem_1, sem_2, sem_3):
    c = lax.axis_index("c")
    s = lax.axis_index("s")
    wid = s * NC + c
    l0 = wid * LW

    pltpu.sync_copy(lengths_hbm, len_v)
    pltpu.sync_copy(w_hbm, wv)
    for b in range(B):
        pltpu.sync_copy(tokens_hbm.at[pl.ds(b * L + l0, LW)], tok_v.at[b])
        pltpu.sync_copy(trees_hbm.at[pl.ds((b * L + l0) * TREE, LW * TREE)],
                        trees_v.at[b])

    lens = len_v[...]
    iota = lax.iota(jnp.int32, LANES)
    keys = iota - lens * LANES
    _, order = plsc.sort_key_val(keys, iota)
    lens_s = plsc.load_gather(len_v, [order])

    zero = jnp.full((LANES,), 0, jnp.int32)
    for g in range(N_G):
        for cc in range(G_ROWS // LANES):
            idx_buf[g, pl.ds(cc * LANES, LANES)] = zero

    def prep(t, carry):
        l_glob = l0 + t
        g = t // G_TS
        col = (t % G_TS) * B
        tcol = jnp.full((LANES,), t, jnp.int32)
        tok = plsc.load_gather(tok_v, [order, tcol])
        valid = jnp.full((LANES,), l_glob, jnp.int32) < lens_s
        idx_buf[g, pl.ds(col, B)] = jnp.clip(jnp.where(valid, tok, 0), 0, 0)
        m_buf[pl.ds(t * B, B)] = jnp.where(valid,
                                           jnp.full((LANES,), 1.0, jnp.float32),
                                           jnp.full((LANES,), 0.0, jnp.float32))
        for k in range(TREE):
            ck = plsc.load_gather(
                trees_v, [order, jnp.full((LANES,), t * TREE + k, jnp.int32)])
            coef_buf[k, pl.ds(t * B, B)] = ck
        return carry

    lax.fori_loop(0, LW, prep, 0)

    bufs = (buf_0, buf_1, buf_2, buf_3)
    sems = (sem_0, sem_1, sem_2, sem_3)

    def dma(g, q):
        return pltpu.make_async_copy(table_hbm.at[idx_buf.at[g]],
                                     bufs[q], sems[q])

    def compute(q, g):
        buf = bufs[q]
        rbase = g * G_ROWS

        def row(j, rcarry, buf=buf):
            jj = rbase + j
            jv = jnp.full((LANES,), jj, jnp.int32)
            mv = plsc.load_gather(m_buf, [jv])
            cvs = [plsc.load_gather(
                       coef_buf, [jnp.full((LANES,), k, jnp.int32), jv])
                   for k in range(TREE)]

            def chunk(u, ccarry, buf=buf):
                sl = pl.ds(u * LANES, LANES)
                acc = buf[j, sl]
                for k in range(TREE):
                    acc = acc + cvs[k] * wv[k, sl]
                buf[j, sl] = acc * mv
                return ccarry

            lax.fori_loop(0, CHUNKS, chunk, 0)
            return rcarry

        lax.fori_loop(0, G_ROWS, row, 0)
        pltpu.sync_copy(buf,
                        out_hbm.at[pl.ds(wid * ROWS_W + rbase, G_ROWS)])

    for q in range(NBUF - 1):
        dma(q, q).start()

    def ring(i, carry):
        gbase = NBUF * i
        for q in range(NBUF):
            g = gbase + q
            dma(g, q).wait()

            @pl.when(g + NBUF - 1 < N_G)
            def _start_next(g=g, q=q):
                dma(g + NBUF - 1, (q + NBUF - 1) % NBUF).start()

            compute(q, g)
        return carry

    lax.fori_loop(0, N_G // NBUF, ring, 0)


def kernel(tokens, trees, lengths, emb_table, W_tree):
    mesh = plsc.VectorSubcoreMesh(core_axis_name="c", subcore_axis_name="s")
    run = pl.kernel(
        _body,
        mesh=mesh,
        compiler_params=pltpu.CompilerParams(needs_layout_passes=False,
                                             use_tc_tiling_on_sc=True),
        out_type=jax.ShapeDtypeStruct((L * B, D), jnp.float32),
        scratch_types=[
            pltpu.VMEM((LANES,), jnp.int32),
            pltpu.VMEM((LANES,), jnp.int32),
            pltpu.VMEM((B, LW), jnp.int32),
            pltpu.VMEM((B, LW * TREE), jnp.float32),
            pltpu.VMEM((TREE, D), jnp.float32),
            pltpu.VMEM((N_G, G_ROWS), jnp.int32),
            pltpu.VMEM((ROWS_W,), jnp.float32),
            pltpu.VMEM((TREE, ROWS_W), jnp.float32),
            pltpu.VMEM((G_ROWS, D), jnp.float32),
            pltpu.VMEM((G_ROWS, D), jnp.float32),
            pltpu.VMEM((G_ROWS, D), jnp.float32),
            pltpu.VMEM((G_ROWS, D), jnp.float32),
            pltpu.SemaphoreType.DMA,
            pltpu.SemaphoreType.DMA,
            pltpu.SemaphoreType.DMA,
            pltpu.SemaphoreType.DMA,
        ],
    )
    flat = run(tokens.astype(jnp.int32).reshape(B * L),
               trees.reshape(B * L * TREE), lengths.astype(jnp.int32),
               emb_table, W_tree)
    return flat.reshape(L, B, D)


# speedup vs baseline: 3.3581x; 3.3581x over previous
import jax
import jax.numpy as jnp
from jax import lax
from jax.experimental import pallas as pl
from jax.experimental.pallas import tpu as pltpu
from jax.experimental.pallas import tpu_sc as plsc

B, L, V, D, TREE = 16, 2048, 50000, 512, 4
NC, NS, LANES = 2, 16, 16
NW = NC * NS
LW = L // NW
ROWS_W = LW * B
G_TS = 2
G_ROWS = G_TS * B
N_G = LW // G_TS
NBUF = 4
CHUNKS = D // LANES


def _body(tokens_hbm, trees_hbm, lengths_hbm, table_hbm, w_hbm, out_hbm,
          len_v, key_buf, tok_v, trees_v, wv, idx_buf, m_buf, coef_buf,
          buf_0, buf_1, buf_2, buf_3, sem_0, sem_1, sem_2, sem_3):
    c = lax.axis_index("c")
    s = lax.axis_index("s")
    wid = s * NC + c
    l0 = wid * LW

    pltpu.sync_copy(lengths_hbm, len_v)
    pltpu.sync_copy(w_hbm, wv)
    for b in range(B):
        pltpu.sync_copy(tokens_hbm.at[pl.ds(b * L + l0, LW)], tok_v.at[b])
        pltpu.sync_copy(trees_hbm.at[pl.ds((b * L + l0) * TREE, LW * TREE)],
                        trees_v.at[b])

    lens = len_v[...]
    iota = lax.iota(jnp.int32, LANES)
    keys = iota - lens * LANES
    _, order = plsc.sort_key_val(keys, iota)
    lens_s = plsc.load_gather(len_v, [order])

    zero = jnp.full((LANES,), 0, jnp.int32)
    for g in range(N_G):
        for cc in range(G_ROWS // LANES):
            idx_buf[g, pl.ds(cc * LANES, LANES)] = zero

    def prep(t, carry):
        l_glob = l0 + t
        g = t // G_TS
        col = (t % G_TS) * B
        tcol = jnp.full((LANES,), t, jnp.int32)
        tok = plsc.load_gather(tok_v, [order, tcol])
        valid = jnp.full((LANES,), l_glob, jnp.int32) < lens_s
        spread = jnp.full((LANES,), wid * ROWS_W + t * B, jnp.int32) + iota
        idx_buf[g, pl.ds(col, B)] = jnp.clip(jnp.where(valid, tok, spread),
                                             0, V - 1)
        m_buf[pl.ds(t * B, B)] = jnp.where(valid,
                                           jnp.full((LANES,), 1.0, jnp.float32),
                                           jnp.full((LANES,), 0.0, jnp.float32))
        for k in range(TREE):
            ck = plsc.load_gather(
                trees_v, [order, jnp.full((LANES,), t * TREE + k, jnp.int32)])
            coef_buf[k, pl.ds(t * B, B)] = ck
        return carry

    lax.fori_loop(0, LW, prep, 0)

    bufs = (buf_0, buf_1, buf_2, buf_3)
    sems = (sem_0, sem_1, sem_2, sem_3)

    def dma(g, q):
        return pltpu.make_async_copy(table_hbm.at[idx_buf.at[g]],
                                     bufs[q], sems[q])

    def compute(q, g):
        buf = bufs[q]
        rbase = g * G_ROWS

        def row(j, rcarry, buf=buf):
            jj = rbase + j
            jv = jnp.full((LANES,), jj, jnp.int32)
            mv = plsc.load_gather(m_buf, [jv])
            cvs = [plsc.load_gather(
                       coef_buf, [jnp.full((LANES,), k, jnp.int32), jv])
                   for k in range(TREE)]

            def chunk(u, ccarry, buf=buf):
                sl = pl.ds(u * LANES, LANES)
                acc = buf[j, sl]
                for k in range(TREE):
                    acc = acc + cvs[k] * wv[k, sl]
                buf[j, sl] = acc * mv
                return ccarry

            lax.fori_loop(0, CHUNKS, chunk, 0)
            return rcarry

        lax.fori_loop(0, G_ROWS, row, 0)
        pltpu.sync_copy(buf,
                        out_hbm.at[pl.ds(wid * ROWS_W + rbase, G_ROWS)])

    for q in range(NBUF - 1):
        dma(q, q).start()

    def ring(i, carry):
        gbase = NBUF * i
        for q in range(NBUF):
            g = gbase + q
            dma(g, q).wait()

            @pl.when(g + NBUF - 1 < N_G)
            def _start_next(g=g, q=q):
                dma(g + NBUF - 1, (q + NBUF - 1) % NBUF).start()

            compute(q, g)
        return carry

    lax.fori_loop(0, N_G // NBUF, ring, 0)


def kernel(tokens, trees, lengths, emb_table, W_tree):
    mesh = plsc.VectorSubcoreMesh(core_axis_name="c", subcore_axis_name="s")
    run = pl.kernel(
        _body,
        mesh=mesh,
        compiler_params=pltpu.CompilerParams(needs_layout_passes=False,
                                             use_tc_tiling_on_sc=True),
        out_type=jax.ShapeDtypeStruct((L * B, D), jnp.float32),
        scratch_types=[
            pltpu.VMEM((LANES,), jnp.int32),
            pltpu.VMEM((LANES,), jnp.int32),
            pltpu.VMEM((B, LW), jnp.int32),
            pltpu.VMEM((B, LW * TREE), jnp.float32),
            pltpu.VMEM((TREE, D), jnp.float32),
            pltpu.VMEM((N_G, G_ROWS), jnp.int32),
            pltpu.VMEM((ROWS_W,), jnp.float32),
            pltpu.VMEM((TREE, ROWS_W), jnp.float32),
            pltpu.VMEM((G_ROWS, D), jnp.float32),
            pltpu.VMEM((G_ROWS, D), jnp.float32),
            pltpu.VMEM((G_ROWS, D), jnp.float32),
            pltpu.VMEM((G_ROWS, D), jnp.float32),
            pltpu.SemaphoreType.DMA,
            pltpu.SemaphoreType.DMA,
            pltpu.SemaphoreType.DMA,
            pltpu.SemaphoreType.DMA,
        ],
    )
    flat = run(tokens.astype(jnp.int32).reshape(B * L),
               trees.reshape(B * L * TREE), lengths.astype(jnp.int32),
               emb_table, W_tree)
    return flat.reshape(L, B, D)


# speedup vs baseline: 3.4771x; 1.0355x over previous
import jax
import jax.numpy as jnp
from jax import lax
from jax.experimental import pallas as pl
from jax.experimental.pallas import tpu as pltpu
from jax.experimental.pallas import tpu_sc as plsc

B, L, V, D, TREE = 16, 2048, 50000, 512, 4
NC, NS, LANES = 2, 16, 16
NW = NC * NS
LW = L // NW
ROWS_W = LW * B
G_TS = 2
G_ROWS = G_TS * B
N_G = LW // G_TS
NBUF = 4
CHUNKS = D // LANES


def _body(tokens_hbm, trees_hbm, lengths_hbm, table_hbm, w_hbm, out_hbm,
          len_v, key_buf, tok_v, trees_v, wv, idx_buf, m_buf, coef_buf,
          buf_0, buf_1, buf_2, buf_3, sem_0, sem_1, sem_2, sem_3):
    c = lax.axis_index("c")
    s = lax.axis_index("s")
    wid = s * NC + c
    l0 = wid * LW

    staged = [pltpu.make_async_copy(lengths_hbm, len_v, sem_0),
              pltpu.make_async_copy(w_hbm, wv, sem_0)]
    for b in range(B):
        staged.append(pltpu.make_async_copy(
            tokens_hbm.at[pl.ds(b * L + l0, LW)], tok_v.at[b], sem_0))
        staged.append(pltpu.make_async_copy(
            trees_hbm.at[pl.ds((b * L + l0) * TREE, LW * TREE)],
            trees_v.at[b], sem_0))
    for cp in staged:
        cp.start()
    for cp in staged:
        cp.wait()

    lens = len_v[...]
    iota = lax.iota(jnp.int32, LANES)
    keys = iota - lens * LANES
    _, order = plsc.sort_key_val(keys, iota)
    lens_s = plsc.load_gather(len_v, [order])

    zero = jnp.full((LANES,), 0, jnp.int32)
    for g in range(N_G):
        for cc in range(G_ROWS // LANES):
            idx_buf[g, pl.ds(cc * LANES, LANES)] = zero

    def prep(t, carry):
        l_glob = l0 + t
        g = t // G_TS
        col = (t % G_TS) * B
        tcol = jnp.full((LANES,), t, jnp.int32)
        tok = plsc.load_gather(tok_v, [order, tcol])
        valid = jnp.full((LANES,), l_glob, jnp.int32) < lens_s
        spread = jnp.full((LANES,), wid * ROWS_W + t * B, jnp.int32) + iota
        idx_buf[g, pl.ds(col, B)] = jnp.clip(jnp.where(valid, tok, spread),
                                             0, V - 1)
        m_buf[pl.ds(t * B, B)] = jnp.where(valid,
                                           jnp.full((LANES,), 1.0, jnp.float32),
                                           jnp.full((LANES,), 0.0, jnp.float32))
        for k in range(TREE):
            ck = plsc.load_gather(
                trees_v, [order, jnp.full((LANES,), t * TREE + k, jnp.int32)])
            coef_buf[k, pl.ds(t * B, B)] = ck
        return carry

    lax.fori_loop(0, LW, prep, 0)

    bufs = (buf_0, buf_1, buf_2, buf_3)
    sems = (sem_0, sem_1, sem_2, sem_3)

    def dma(g, q):
        return pltpu.make_async_copy(table_hbm.at[idx_buf.at[g]],
                                     bufs[q], sems[q])

    def compute(q, g):
        buf = bufs[q]
        rbase = g * G_ROWS

        def row(j, rcarry, buf=buf):
            jj = rbase + j
            jv = jnp.full((LANES,), jj, jnp.int32)
            mv = plsc.load_gather(m_buf, [jv])
            cvs = [plsc.load_gather(
                       coef_buf, [jnp.full((LANES,), k, jnp.int32), jv])
                   for k in range(TREE)]

            def chunk(u, ccarry, buf=buf):
                sl = pl.ds(u * LANES, LANES)
                acc = buf[j, sl]
                for k in range(TREE):
                    acc = acc + cvs[k] * wv[k, sl]
                buf[j, sl] = acc * mv
                return ccarry

            lax.fori_loop(0, CHUNKS, chunk, 0)
            return rcarry

        lax.fori_loop(0, G_ROWS, row, 0)
        pltpu.sync_copy(buf,
                        out_hbm.at[pl.ds(wid * ROWS_W + rbase, G_ROWS)])

    for q in range(NBUF - 1):
        dma(q, q).start()

    def ring(i, carry):
        gbase = NBUF * i
        for q in range(NBUF):
            g = gbase + q
            dma(g, q).wait()

            @pl.when(g + NBUF - 1 < N_G)
            def _start_next(g=g, q=q):
                dma(g + NBUF - 1, (q + NBUF - 1) % NBUF).start()

            compute(q, g)
        return carry

    lax.fori_loop(0, N_G // NBUF, ring, 0)


def kernel(tokens, trees, lengths, emb_table, W_tree):
    mesh = plsc.VectorSubcoreMesh(core_axis_name="c", subcore_axis_name="s")
    run = pl.kernel(
        _body,
        mesh=mesh,
        compiler_params=pltpu.CompilerParams(needs_layout_passes=False,
                                             use_tc_tiling_on_sc=True),
        out_type=jax.ShapeDtypeStruct((L * B, D), jnp.float32),
        scratch_types=[
            pltpu.VMEM((LANES,), jnp.int32),
            pltpu.VMEM((LANES,), jnp.int32),
            pltpu.VMEM((B, LW), jnp.int32),
            pltpu.VMEM((B, LW * TREE), jnp.float32),
            pltpu.VMEM((TREE, D), jnp.float32),
            pltpu.VMEM((N_G, G_ROWS), jnp.int32),
            pltpu.VMEM((ROWS_W,), jnp.float32),
            pltpu.VMEM((TREE, ROWS_W), jnp.float32),
            pltpu.VMEM((G_ROWS, D), jnp.float32),
            pltpu.VMEM((G_ROWS, D), jnp.float32),
            pltpu.VMEM((G_ROWS, D), jnp.float32),
            pltpu.VMEM((G_ROWS, D), jnp.float32),
            pltpu.SemaphoreType.DMA,
            pltpu.SemaphoreType.DMA,
            pltpu.SemaphoreType.DMA,
            pltpu.SemaphoreType.DMA,
        ],
    )
    flat = run(tokens.astype(jnp.int32).reshape(B * L),
               trees.reshape(B * L * TREE), lengths.astype(jnp.int32),
               emb_table, W_tree)
    return flat.reshape(L, B, D)


# speedup vs baseline: 3.4941x; 1.0049x over previous
import jax
import jax.numpy as jnp
from jax import lax
from jax.experimental import pallas as pl
from jax.experimental.pallas import tpu as pltpu
from jax.experimental.pallas import tpu_sc as plsc

B, L, V, D, TREE = 16, 2048, 50000, 512, 4
NC, NS, LANES = 2, 16, 16
NW = NC * NS
LW = L // NW
ROWS_W = LW * B
G_TS = 4
G_ROWS = G_TS * B
N_G = LW // G_TS
NBUF = 2
CHUNKS = D // LANES


def _body(tokens_hbm, trees_hbm, lengths_hbm, table_hbm, w_hbm, out_hbm,
          len_v, key_buf, tok_v, trees_v, wv, idx_buf, m_buf, coef_buf,
          buf_0, buf_1, sem_0, sem_1):
    c = lax.axis_index("c")
    s = lax.axis_index("s")
    wid = s * NC + c
    l0 = wid * LW

    staged = [pltpu.make_async_copy(lengths_hbm, len_v, sem_0),
              pltpu.make_async_copy(w_hbm, wv, sem_0)]
    for b in range(B):
        staged.append(pltpu.make_async_copy(
            tokens_hbm.at[pl.ds(b * L + l0, LW)], tok_v.at[b], sem_0))
        staged.append(pltpu.make_async_copy(
            trees_hbm.at[pl.ds((b * L + l0) * TREE, LW * TREE)],
            trees_v.at[b], sem_0))
    for cp in staged:
        cp.start()
    for cp in staged:
        cp.wait()

    lens = len_v[...]
    iota = lax.iota(jnp.int32, LANES)
    keys = iota - lens * LANES
    _, order = plsc.sort_key_val(keys, iota)
    lens_s = plsc.load_gather(len_v, [order])

    zero = jnp.full((LANES,), 0, jnp.int32)
    for g in range(N_G):
        for cc in range(G_ROWS // LANES):
            idx_buf[g, pl.ds(cc * LANES, LANES)] = zero

    def prep(t, carry):
        l_glob = l0 + t
        g = t // G_TS
        col = (t % G_TS) * B
        tcol = jnp.full((LANES,), t, jnp.int32)
        tok = plsc.load_gather(tok_v, [order, tcol])
        valid = jnp.full((LANES,), l_glob, jnp.int32) < lens_s
        spread = jnp.full((LANES,), wid * ROWS_W + t * B, jnp.int32) + iota
        idx_buf[g, pl.ds(col, B)] = jnp.clip(jnp.where(valid, tok, spread),
                                             0, V - 1)
        m_buf[pl.ds(t * B, B)] = jnp.where(valid,
                                           jnp.full((LANES,), 1.0, jnp.float32),
                                           jnp.full((LANES,), 0.0, jnp.float32))
        for k in range(TREE):
            ck = plsc.load_gather(
                trees_v, [order, jnp.full((LANES,), t * TREE + k, jnp.int32)])
            coef_buf[k, pl.ds(t * B, B)] = ck
        return carry

    lax.fori_loop(0, LW, prep, 0)

    bufs = (buf_0, buf_1)
    sems = (sem_0, sem_1)

    def dma(g, q):
        return pltpu.make_async_copy(table_hbm.at[idx_buf.at[g]],
                                     bufs[q], sems[q])

    def compute(q, g):
        buf = bufs[q]
        rbase = g * G_ROWS

        def row(j, rcarry, buf=buf):
            jj = rbase + j
            jv = jnp.full((LANES,), jj, jnp.int32)
            mv = plsc.load_gather(m_buf, [jv])
            cvs = [plsc.load_gather(
                       coef_buf, [jnp.full((LANES,), k, jnp.int32), jv])
                   for k in range(TREE)]

            def chunk(u, ccarry, buf=buf):
                sl = pl.ds(u * LANES, LANES)
                acc = buf[j, sl]
                for k in range(TREE):
                    acc = acc + cvs[k] * wv[k, sl]
                buf[j, sl] = acc * mv
                return ccarry

            lax.fori_loop(0, CHUNKS, chunk, 0)
            return rcarry

        lax.fori_loop(0, G_ROWS, row, 0)
        pltpu.sync_copy(buf,
                        out_hbm.at[pl.ds(wid * ROWS_W + rbase, G_ROWS)])

    for q in range(NBUF - 1):
        dma(q, q).start()

    def ring(i, carry):
        gbase = NBUF * i
        for q in range(NBUF):
            g = gbase + q
            dma(g, q).wait()

            @pl.when(g + NBUF - 1 < N_G)
            def _start_next(g=g, q=q):
                dma(g + NBUF - 1, (q + NBUF - 1) % NBUF).start()

            compute(q, g)
        return carry

    lax.fori_loop(0, N_G // NBUF, ring, 0)


def kernel(tokens, trees, lengths, emb_table, W_tree):
    mesh = plsc.VectorSubcoreMesh(core_axis_name="c", subcore_axis_name="s")
    run = pl.kernel(
        _body,
        mesh=mesh,
        compiler_params=pltpu.CompilerParams(needs_layout_passes=False,
                                             use_tc_tiling_on_sc=True),
        out_type=jax.ShapeDtypeStruct((L * B, D), jnp.float32),
        scratch_types=[
            pltpu.VMEM((LANES,), jnp.int32),
            pltpu.VMEM((LANES,), jnp.int32),
            pltpu.VMEM((B, LW), jnp.int32),
            pltpu.VMEM((B, LW * TREE), jnp.float32),
            pltpu.VMEM((TREE, D), jnp.float32),
            pltpu.VMEM((N_G, G_ROWS), jnp.int32),
            pltpu.VMEM((ROWS_W,), jnp.float32),
            pltpu.VMEM((TREE, ROWS_W), jnp.float32),
            pltpu.VMEM((G_ROWS, D), jnp.float32),
            pltpu.VMEM((G_ROWS, D), jnp.float32),
            pltpu.SemaphoreType.DMA,
            pltpu.SemaphoreType.DMA,
        ],
    )
    flat = run(tokens.astype(jnp.int32).reshape(B * L),
               trees.reshape(B * L * TREE), lengths.astype(jnp.int32),
               emb_table, W_tree)
    return flat.reshape(L, B, D)
